# Initial kernel scaffold; baseline (speedup 1.0000x reference)
import functools
import jax
import jax.numpy as jnp
from jax import lax
from jax.experimental import pallas as pl
from jax.experimental.pallas import tpu as pltpu
from jax.experimental.pallas import tpu_sc as plsc

NC = 2
NS = 16
NW = NC * NS
LANES = 16
K = 125


def _fill_zeros(ref, nrows, nlane_blocks):
    def body(t, _):
        i = t // nlane_blocks
        j = t % nlane_blocks
        ref[i, pl.ds(j * LANES, LANES)] = jnp.zeros((LANES,), jnp.float32)
        return 0
    lax.fori_loop(0, nrows * nlane_blocks, body, 0)


def _make_deg_kernel(n_nodes, nchunk):
    rows_per = n_nodes // NS
    mesh = plsc.VectorSubcoreMesh(core_axis_name="c", subcore_axis_name="s")

    @functools.partial(
        pl.kernel,
        out_type=jax.ShapeDtypeStruct((NC, n_nodes, LANES), jnp.float32),
        mesh=mesh,
        scratch_types=[
            pltpu.VMEM((nchunk, K), jnp.int32),
            pltpu.VMEM((K, LANES), jnp.float32),
            pltpu.VMEM((rows_per, LANES), jnp.float32),
            pltpu.VMEM_SHARED((n_nodes, LANES), jnp.float32),
        ],
    )
    def deg_kernel(dst_hbm, out_hbm, dst_loc, ones_v, zbuf, acc):
        cid = lax.axis_index("c")
        sid = lax.axis_index("s")
        wid = cid * NS + sid
        pltpu.sync_copy(dst_hbm.at[wid], dst_loc)

        def fill_ones(i, _):
            ones_v[i, :] = jnp.ones((LANES,), jnp.float32)
            return 0
        lax.fori_loop(0, K, fill_ones, 0)
        _fill_zeros(zbuf, rows_per, 1)
        pltpu.sync_copy(zbuf, acc.at[pl.ds(sid * rows_per, rows_per)])
        plsc.subcore_barrier()

        def body(j, _):
            pltpu.sync_copy(ones_v, acc.at[dst_loc.at[j]], add=True)
            return 0
        lax.fori_loop(0, nchunk, body, 0)
        plsc.subcore_barrier()
        pltpu.sync_copy(acc.at[pl.ds(sid * rows_per, rows_per)],
                        out_hbm.at[cid, pl.ds(sid * rows_per, rows_per)])

    return deg_kernel


def _make_agg_kernel(n_nodes, d, nchunk):
    rows_per = n_nodes // NS
    zr = 125
    mesh = plsc.VectorSubcoreMesh(core_axis_name="c", subcore_axis_name="s")

    @functools.partial(
        pl.kernel,
        out_type=jax.ShapeDtypeStruct((NC, n_nodes, d), jnp.float32),
        mesh=mesh,
        scratch_types=[
            pltpu.VMEM((nchunk, K), jnp.int32),
            pltpu.VMEM((nchunk, K), jnp.int32),
            pltpu.VMEM((K, d), jnp.float32),
            pltpu.VMEM((zr, d), jnp.float32),
            pltpu.VMEM_SHARED((n_nodes, d), jnp.float32),
            pltpu.SemaphoreType.DMA,
        ],
    )
    def agg_kernel(hs_hbm, src_hbm, dst_hbm, out_hbm,
                   src_loc, dst_loc, rows, zbuf, acc, sem):
        cid = lax.axis_index("c")
        sid = lax.axis_index("s")
        wid = cid * NS + sid
        pltpu.sync_copy(src_hbm.at[wid], src_loc)
        pltpu.sync_copy(dst_hbm.at[wid], dst_loc)

        _fill_zeros(zbuf, zr, d // LANES)
        for q in range(rows_per // zr):
            pltpu.sync_copy(zbuf, acc.at[pl.ds(sid * rows_per + q * zr, zr)])
        plsc.subcore_barrier()

        def body(j, _):
            pltpu.async_copy(hs_hbm.at[src_loc.at[j]], rows, sem).wait()
            pltpu.sync_copy(rows, acc.at[dst_loc.at[j]], add=True)
            return 0
        lax.fori_loop(0, nchunk, body, 0)
        plsc.subcore_barrier()
        pltpu.sync_copy(acc.at[pl.ds(sid * rows_per, rows_per)],
                        out_hbm.at[cid, pl.ds(sid * rows_per, rows_per)])

    return agg_kernel


def _tc1(x, w1, degp):
    n, d_out = x.shape[0], w1.shape[1]

    def body(x_ref, w_ref, deg_ref, hs_ref, dis_ref):
        deg = deg_ref[0, :, :] + deg_ref[1, :, :]
        dis = lax.rsqrt(deg[:, 0:1] + 1.0)
        h = jnp.dot(x_ref[...], w_ref[...],
                    preferred_element_type=jnp.float32)
        hs_ref[...] = dis * h
        dis_ref[...] = dis

    return pl.pallas_call(
        body,
        out_shape=(jax.ShapeDtypeStruct((n, d_out), jnp.float32),
                   jax.ShapeDtypeStruct((n, 1), jnp.float32)),
    )(x, w1, degp)


def _tc2(agg, hs1, dis, b1, w2):
    n, d_out = hs1.shape[0], w2.shape[1]

    def body(agg_ref, hs_ref, dis_ref, b_ref, w_ref, out_ref):
        s = agg_ref[0, :, :] + agg_ref[1, :, :] + hs_ref[...]
        z = jnp.maximum(dis_ref[...] * s + b_ref[...], 0.0)
        h2 = jnp.dot(z, w_ref[...], preferred_element_type=jnp.float32)
        out_ref[...] = dis_ref[...] * h2

    return pl.pallas_call(
        body,
        out_shape=jax.ShapeDtypeStruct((n, d_out), jnp.float32),
    )(agg, hs1, dis, b1, w2)


def _tc3(agg, hs2, dis, b2, wfc, bfc):
    n, d_out = hs2.shape[0], wfc.shape[1]

    def body(agg_ref, hs_ref, dis_ref, b_ref, w_ref, bfc_ref, out_ref):
        s = agg_ref[0, :, :] + agg_ref[1, :, :] + hs_ref[...]
        z = jnp.maximum(dis_ref[...] * s + b_ref[...], 0.0)
        out_ref[...] = jnp.dot(z, w_ref[...],
                               preferred_element_type=jnp.float32) + bfc_ref[...]

    return pl.pallas_call(
        body,
        out_shape=jax.ShapeDtypeStruct((n, d_out), jnp.float32),
    )(agg, hs2, dis, b2, wfc, bfc)


def kernel(x, edge_index, W1, b1, W2, b2, Wfc, bfc):
    n = x.shape[0]
    e = edge_index.shape[1]
    hid2 = W1.shape[1]
    hid = W2.shape[1]
    ncls = Wfc.shape[1]

    assert e % (NW * K) == 0 and n % NS == 0
    nchunk = e // (NW * K)

    ei = edge_index.astype(jnp.int32)
    src3 = ei[0].reshape(NW, nchunk, K)
    dst3 = ei[1].reshape(NW, nchunk, K)

    wfc_p = jnp.pad(Wfc, ((0, 0), (0, 128 - ncls)))
    bfc_p = jnp.pad(bfc, (0, 128 - ncls)).reshape(1, 128)
    b1r = b1.reshape(1, hid2)
    b2r = b2.reshape(1, hid)

    degp = _make_deg_kernel(n, nchunk)(dst3)
    hs1, dis = _tc1(x, W1, degp)
    agg1 = _make_agg_kernel(n, hid2, nchunk)(hs1, src3, dst3)
    hs2 = _tc2(agg1, hs1, dis, b1r, W2)
    agg2 = _make_agg_kernel(n, hid, nchunk)(hs2, src3, dst3)
    out = _tc3(agg2, hs2, dis, b2r, wfc_p, bfc_p)
    return out[:, :ncls]


# speedup vs baseline: 20.5984x; 20.5984x over previous
import functools
import jax
import jax.numpy as jnp
from jax import lax
from jax.experimental import pallas as pl
from jax.experimental.pallas import tpu as pltpu
from jax.experimental.pallas import tpu_sc as plsc

NC = 2
NS = 16
NW = NC * NS
D = 128
K = 125
ZR = 32


def _fill_zeros(ref, nrows):
    def body(t, _):
        i = t // (D // 16)
        j = t % (D // 16)
        ref[i, pl.ds(j * 16, 16)] = jnp.zeros((16,), jnp.float32)
        return 0
    lax.fori_loop(0, nrows * (D // 16), body, 0)


def _make_deg_kernel(n_nodes, nchunk):
    rows_per = n_nodes // NS
    mesh = plsc.VectorSubcoreMesh(core_axis_name="c", subcore_axis_name="s")

    @functools.partial(
        pl.kernel,
        out_type=jax.ShapeDtypeStruct((NC, n_nodes, D), jnp.float32),
        mesh=mesh,
        scratch_types=[
            pltpu.VMEM((nchunk, K), jnp.int32),
            pltpu.VMEM((K, D), jnp.float32),
            pltpu.VMEM((ZR, D), jnp.float32),
            pltpu.VMEM_SHARED((n_nodes, D), jnp.float32),
        ],
    )
    def deg_kernel(dst_hbm, out_hbm, dst_loc, ones_v, zbuf, acc):
        cid = lax.axis_index("c")
        sid = lax.axis_index("s")
        wid = cid * NS + sid
        pltpu.sync_copy(dst_hbm.at[wid], dst_loc)

        def fill_ones(t, _):
            i = t // (D // 16)
            j = t % (D // 16)
            ones_v[i, pl.ds(j * 16, 16)] = jnp.ones((16,), jnp.float32)
            return 0
        lax.fori_loop(0, K * (D // 16), fill_ones, 0)
        _fill_zeros(zbuf, ZR)
        for q in range(rows_per // ZR):
            pltpu.sync_copy(zbuf, acc.at[pl.ds(sid * rows_per + q * ZR, ZR)])
        plsc.subcore_barrier()

        def body(j, _):
            pltpu.sync_copy(ones_v, acc.at[dst_loc.at[j]], add=True)
            return 0
        lax.fori_loop(0, nchunk, body, 0)
        plsc.subcore_barrier()
        for q in range(rows_per // ZR):
            r = sid * rows_per + q * ZR
            pltpu.sync_copy(acc.at[pl.ds(r, ZR)], zbuf)
            pltpu.sync_copy(zbuf, out_hbm.at[cid, pl.ds(r, ZR)])

    return deg_kernel


def _make_agg_kernel(n_nodes, nchunk):
    rows_per = n_nodes // NS
    mesh = plsc.VectorSubcoreMesh(core_axis_name="c", subcore_axis_name="s")

    @functools.partial(
        pl.kernel,
        out_type=jax.ShapeDtypeStruct((NC, n_nodes, D), jnp.float32),
        mesh=mesh,
        scratch_types=[
            pltpu.VMEM((nchunk, K), jnp.int32),
            pltpu.VMEM((nchunk, K), jnp.int32),
            pltpu.VMEM((K, D), jnp.float32),
            pltpu.VMEM((ZR, D), jnp.float32),
            pltpu.VMEM_SHARED((n_nodes, D), jnp.float32),
            pltpu.SemaphoreType.DMA,
        ],
    )
    def agg_kernel(hs_hbm, src_hbm, dst_hbm, out_hbm,
                   src_loc, dst_loc, rows, zbuf, acc, sem):
        cid = lax.axis_index("c")
        sid = lax.axis_index("s")
        wid = cid * NS + sid
        pltpu.sync_copy(src_hbm.at[wid], src_loc)
        pltpu.sync_copy(dst_hbm.at[wid], dst_loc)

        _fill_zeros(zbuf, ZR)
        for q in range(rows_per // ZR):
            pltpu.sync_copy(zbuf, acc.at[pl.ds(sid * rows_per + q * ZR, ZR)])
        plsc.subcore_barrier()

        def body(j, _):
            pltpu.async_copy(hs_hbm.at[src_loc.at[j]], rows, sem).wait()
            pltpu.sync_copy(rows, acc.at[dst_loc.at[j]], add=True)
            return 0
        lax.fori_loop(0, nchunk, body, 0)
        plsc.subcore_barrier()
        for q in range(rows_per // ZR):
            r = sid * rows_per + q * ZR
            pltpu.sync_copy(acc.at[pl.ds(r, ZR)], zbuf)
            pltpu.sync_copy(zbuf, out_hbm.at[cid, pl.ds(r, ZR)])

    return agg_kernel


def _tc1(x, w1, degp):
    n, d_out = x.shape[0], w1.shape[1]

    def body(x_ref, w_ref, deg_ref, hs_ref, dis_ref):
        deg = deg_ref[0, :, 0:1] + deg_ref[1, :, 0:1]
        dis = lax.rsqrt(deg + 1.0)
        h = jnp.dot(x_ref[...], w_ref[...],
                    preferred_element_type=jnp.float32)
        hs_ref[...] = dis * h
        dis_ref[...] = dis

    return pl.pallas_call(
        body,
        out_shape=(jax.ShapeDtypeStruct((n, d_out), jnp.float32),
                   jax.ShapeDtypeStruct((n, 1), jnp.float32)),
    )(x, w1, degp)


def _tc2(agg, hs1, dis, b1, w2):
    n, d_out = hs1.shape[0], w2.shape[1]

    def body(agg_ref, hs_ref, dis_ref, b_ref, w_ref, out_ref):
        s = agg_ref[0, :, :] + agg_ref[1, :, :] + hs_ref[...]
        z = jnp.maximum(dis_ref[...] * s + b_ref[...], 0.0)
        h2 = jnp.dot(z, w_ref[...], preferred_element_type=jnp.float32)
        out_ref[...] = dis_ref[...] * h2

    return pl.pallas_call(
        body,
        out_shape=jax.ShapeDtypeStruct((n, d_out), jnp.float32),
    )(agg, hs1, dis, b1, w2)


def _tc3(agg, hs2, dis, b2, wfc, bfc):
    n, d_out = hs2.shape[0], wfc.shape[1]

    def body(agg_ref, hs_ref, dis_ref, b_ref, w_ref, bfc_ref, out_ref):
        s = agg_ref[0, :, :] + agg_ref[1, :, :] + hs_ref[...]
        z = jnp.maximum(dis_ref[...] * s + b_ref[...], 0.0)
        out_ref[...] = jnp.dot(z, w_ref[...],
                               preferred_element_type=jnp.float32) + bfc_ref[...]

    return pl.pallas_call(
        body,
        out_shape=jax.ShapeDtypeStruct((n, d_out), jnp.float32),
    )(agg, hs2, dis, b2, wfc, bfc)


def kernel(x, edge_index, W1, b1, W2, b2, Wfc, bfc):
    n = x.shape[0]
    e = edge_index.shape[1]
    hid2 = W1.shape[1]
    hid = W2.shape[1]
    ncls = Wfc.shape[1]

    npad = ((n + NS * ZR - 1) // (NS * ZR)) * (NS * ZR)
    assert e % (NW * K) == 0 and hid2 == D

    nchunk = e // (NW * K)

    ei = edge_index.astype(jnp.int32)
    src3 = ei[0].reshape(NW, nchunk, K)
    dst3 = ei[1].reshape(NW, nchunk, K)

    x_p = jnp.pad(x, ((0, npad - n), (0, 0)))
    w2_p = jnp.pad(W2, ((0, 0), (0, D - hid)))
    b2r = jnp.pad(b2, (0, D - hid)).reshape(1, D)
    wfc_p = jnp.pad(Wfc, ((0, D - hid), (0, D - ncls)))
    bfc_p = jnp.pad(bfc, (0, D - ncls)).reshape(1, D)
    b1r = b1.reshape(1, hid2)

    degp = _make_deg_kernel(npad, nchunk)(dst3)
    hs1, dis = _tc1(x_p, W1, degp)
    agg1 = _make_agg_kernel(npad, nchunk)(hs1, src3, dst3)
    hs2 = _tc2(agg1, hs1, dis, b1r, w2_p)
    agg2 = _make_agg_kernel(npad, nchunk)(hs2, src3, dst3)
    out = _tc3(agg2, hs2, dis, b2r, wfc_p, bfc_p)
    return out[:n, :ncls]
